# Initial kernel scaffold; baseline (speedup 1.0000x reference)
import jax
import jax.numpy as jnp
from jax.experimental import pallas as pl


def kernel(s, r, o, subj_table, rel_table, obj_table, W, b):
    raise NotImplementedError("write your pallas kernel here")


# speedup vs baseline: 4.5505x; 4.5505x over previous
import functools

import jax
import jax.numpy as jnp
import numpy as np
from jax import lax
from jax.experimental import pallas as pl
from jax.experimental.pallas import tpu as pltpu
from jax.experimental.pallas import tpu_sc as plsc

D = 128
NS_SUBJ = 16
NS_REL = 9
NS_OBJ = 16
N_COMBO = NS_SUBJ * NS_REL * NS_OBJ
STACK = NS_SUBJ + NS_REL + NS_OBJ
STACK_PAD = 48

SC_CORES = 2
SC_SUBCORES = 16
SC_LANES = 16
NW = SC_CORES * SC_SUBCORES


def _build_selector() -> np.ndarray:
    e = np.zeros((N_COMBO, STACK_PAD), np.float32)
    v = np.arange(N_COMBO)
    e[v, v // (NS_REL * NS_OBJ)] = 1.0
    e[v, NS_SUBJ + (v // NS_OBJ) % NS_REL] = 1.0
    e[v, NS_SUBJ + NS_REL + v % NS_OBJ] = 1.0
    return e


_SELECTOR = _build_selector()


def _fold_body(t_ref, wt_ref, e_ref, b_ref, c_ref):
    p = lax.dot_general(t_ref[...], wt_ref[...], (((1,), (0,)), ((), ())),
                        preferred_element_type=jnp.float32)
    c = lax.dot_general(e_ref[...], p, (((1,), (0,)), ((), ())),
                        preferred_element_type=jnp.float32)
    c_ref[...] = c + b_ref[...]


def _build_combined_table(subj_table, rel_table, obj_table, W, b):
    t = jnp.zeros((STACK_PAD, 3 * D), jnp.float32)
    t = t.at[0:NS_SUBJ, 0:D].set(subj_table)
    t = t.at[NS_SUBJ:NS_SUBJ + NS_REL, D:2 * D].set(rel_table)
    t = t.at[NS_SUBJ + NS_REL:STACK, 2 * D:3 * D].set(obj_table)
    return pl.pallas_call(
        _fold_body,
        out_shape=jax.ShapeDtypeStruct((N_COMBO, D), jnp.float32),
    )(t, W.T, jnp.asarray(_SELECTOR), b.reshape(1, D))


def _gather_kernel_fn(B, b_per_w, n_chunks, G):
    mesh = plsc.VectorSubcoreMesh(core_axis_name="c", subcore_axis_name="s")

    @functools.partial(
        pl.kernel,
        mesh=mesh,
        out_type=jax.ShapeDtypeStruct((B, D), jnp.float32),
        scratch_types=[
            pltpu.VMEM((b_per_w,), jnp.int32),
            pltpu.VMEM((b_per_w,), jnp.int32),
            pltpu.VMEM((b_per_w,), jnp.int32),
            pltpu.VMEM((n_chunks, G), jnp.int32),
            pltpu.VMEM((G, D), jnp.float32),
            pltpu.SemaphoreType.DMA,
        ],
    )
    def k(s_hbm, r_hbm, o_hbm, table_hbm, out_hbm,
          s_v, r_v, o_v, j_v, rows_v, sem):
        wid = lax.axis_index("s") * SC_CORES + lax.axis_index("c")
        base = wid * b_per_w
        pltpu.sync_copy(s_hbm.at[pl.ds(base, b_per_w)], s_v)
        pltpu.sync_copy(r_hbm.at[pl.ds(base, b_per_w)], r_v)
        pltpu.sync_copy(o_hbm.at[pl.ds(base, b_per_w)], o_v)

        for c in range(n_chunks):
            @pl.loop(0, G, step=SC_LANES)
            def _(i, c=c):
                src = pl.ds(c * G + i, SC_LANES)
                j_v[c, pl.ds(i, SC_LANES)] = (
                    s_v[src] * (NS_REL * NS_OBJ)
                    + r_v[src] * NS_OBJ
                    + o_v[src])

        for c in range(n_chunks):
            pltpu.async_copy(table_hbm.at[j_v.at[c]], rows_v, sem).wait()
            pltpu.sync_copy(rows_v, out_hbm.at[pl.ds(base + c * G, G)])

    return k


def kernel(s, r, o, subj_table, rel_table, obj_table, W, b):
    B = s.shape[0]
    b_per_w = B // NW
    G = 128
    n_chunks = b_per_w // G

    table = _build_combined_table(subj_table, rel_table, obj_table, W, b)
    gk = _gather_kernel_fn(B, b_per_w, n_chunks, G)
    return gk(s.astype(jnp.int32), r.astype(jnp.int32), o.astype(jnp.int32),
              table)


# speedup vs baseline: 5.8287x; 1.2809x over previous
import functools

import jax
import jax.numpy as jnp
import numpy as np
from jax import lax
from jax.experimental import pallas as pl
from jax.experimental.pallas import tpu as pltpu
from jax.experimental.pallas import tpu_sc as plsc

D = 128
NS_SUBJ = 16
NS_REL = 9
NS_OBJ = 16
N_COMBO = NS_SUBJ * NS_REL * NS_OBJ
STACK = NS_SUBJ + NS_REL + NS_OBJ
STACK_PAD = 48

SC_CORES = 2
SC_SUBCORES = 16
SC_LANES = 16
NW = SC_CORES * SC_SUBCORES


def _build_selectors():
    v = np.arange(N_COMBO)
    es = np.zeros((N_COMBO, NS_SUBJ), np.float32)
    er = np.zeros((N_COMBO, NS_REL), np.float32)
    eo = np.zeros((N_COMBO, NS_OBJ), np.float32)
    es[v, v // (NS_REL * NS_OBJ)] = 1.0
    er[v, (v // NS_OBJ) % NS_REL] = 1.0
    eo[v, v % NS_OBJ] = 1.0
    return es, er, eo


_ES, _ER, _EO = _build_selectors()


def _fold_body(subj_ref, rel_ref, obj_ref, w_ref, es_ref, er_ref, eo_ref,
               b_ref, c_ref):
    dn_t = (((1,), (1,)), ((), ()))
    dn = (((1,), (0,)), ((), ()))
    ps = lax.dot_general(subj_ref[...], w_ref[:, 0:D], dn_t,
                         preferred_element_type=jnp.float32)
    pr = lax.dot_general(rel_ref[...], w_ref[:, D:2 * D], dn_t,
                         preferred_element_type=jnp.float32)
    po = lax.dot_general(obj_ref[...], w_ref[:, 2 * D:3 * D], dn_t,
                         preferred_element_type=jnp.float32)
    c = (lax.dot_general(es_ref[...], ps, dn,
                         preferred_element_type=jnp.float32)
         + lax.dot_general(er_ref[...], pr, dn,
                           preferred_element_type=jnp.float32)
         + lax.dot_general(eo_ref[...], po, dn,
                           preferred_element_type=jnp.float32))
    c_ref[...] = c + b_ref[...]


def _build_combined_table(subj_table, rel_table, obj_table, W, b):
    return pl.pallas_call(
        _fold_body,
        out_shape=jax.ShapeDtypeStruct((N_COMBO, D), jnp.float32),
    )(subj_table, rel_table, obj_table, W,
      jnp.asarray(_ES), jnp.asarray(_ER), jnp.asarray(_EO), b.reshape(1, D))


def _gather_kernel_fn(B, b_per_w, n_chunks, G):
    mesh = plsc.VectorSubcoreMesh(core_axis_name="c", subcore_axis_name="s")

    @functools.partial(
        pl.kernel,
        mesh=mesh,
        out_type=jax.ShapeDtypeStruct((B, D), jnp.float32),
        scratch_types=[
            pltpu.VMEM((b_per_w,), jnp.int32),
            pltpu.VMEM((b_per_w,), jnp.int32),
            pltpu.VMEM((b_per_w,), jnp.int32),
            pltpu.VMEM((n_chunks, G), jnp.int32),
            pltpu.VMEM((n_chunks, G, D), jnp.float32),
            pltpu.SemaphoreType.DMA,
            pltpu.SemaphoreType.DMA,
        ],
    )
    def k(s_hbm, r_hbm, o_hbm, table_hbm, out_hbm,
          s_v, r_v, o_v, j_v, rows_v, gsem, osem):
        wid = lax.axis_index("s") * SC_CORES + lax.axis_index("c")
        base = wid * b_per_w
        pltpu.sync_copy(s_hbm.at[pl.ds(base, b_per_w)], s_v)
        pltpu.sync_copy(r_hbm.at[pl.ds(base, b_per_w)], r_v)
        pltpu.sync_copy(o_hbm.at[pl.ds(base, b_per_w)], o_v)

        for c in range(n_chunks):
            @pl.loop(0, G, step=SC_LANES)
            def _(i, c=c):
                src = pl.ds(c * G + i, SC_LANES)
                j_v[c, pl.ds(i, SC_LANES)] = (
                    s_v[src] * (NS_REL * NS_OBJ)
                    + r_v[src] * NS_OBJ
                    + o_v[src])

        gathers = [
            pltpu.async_copy(table_hbm.at[j_v.at[c]], rows_v.at[c], gsem)
            for c in range(n_chunks)
        ]
        writes = []
        for c in range(n_chunks):
            gathers[c].wait()
            writes.append(pltpu.async_copy(
                rows_v.at[c], out_hbm.at[pl.ds(base + c * G, G)], osem))
        for w in writes:
            w.wait()

    return k


def kernel(s, r, o, subj_table, rel_table, obj_table, W, b):
    B = s.shape[0]
    b_per_w = B // NW
    G = 128
    n_chunks = b_per_w // G

    table = _build_combined_table(subj_table, rel_table, obj_table, W, b)
    gk = _gather_kernel_fn(B, b_per_w, n_chunks, G)
    return gk(s.astype(jnp.int32), r.astype(jnp.int32), o.astype(jnp.int32),
              table)


# speedup vs baseline: 6.6201x; 1.1358x over previous
import functools

import jax
import jax.numpy as jnp
import numpy as np
from jax import lax
from jax.experimental import pallas as pl
from jax.experimental.pallas import tpu as pltpu
from jax.experimental.pallas import tpu_sc as plsc

D = 128
NS_SUBJ = 16
NS_REL = 9
NS_OBJ = 16
N_COMBO = NS_SUBJ * NS_REL * NS_OBJ
STACK = NS_SUBJ + NS_REL + NS_OBJ
STACK_PAD = 48

SC_CORES = 2
SC_SUBCORES = 16
SC_LANES = 16
NW = SC_CORES * SC_SUBCORES


def _build_selectors():
    v = np.arange(N_COMBO)
    es = np.zeros((N_COMBO, NS_SUBJ), np.float32)
    er = np.zeros((N_COMBO, NS_REL), np.float32)
    eo = np.zeros((N_COMBO, NS_OBJ), np.float32)
    es[v, v // (NS_REL * NS_OBJ)] = 1.0
    er[v, (v // NS_OBJ) % NS_REL] = 1.0
    eo[v, v % NS_OBJ] = 1.0
    return es, er, eo


_ES, _ER, _EO = _build_selectors()


def _fold_body(subj_ref, rel_ref, obj_ref, w_ref, es_ref, er_ref, eo_ref,
               b_ref, c_ref):
    dn_t = (((1,), (1,)), ((), ()))
    dn = (((1,), (0,)), ((), ()))
    ps = lax.dot_general(subj_ref[...], w_ref[:, 0:D], dn_t,
                         preferred_element_type=jnp.float32)
    pr = lax.dot_general(rel_ref[...], w_ref[:, D:2 * D], dn_t,
                         preferred_element_type=jnp.float32)
    po = lax.dot_general(obj_ref[...], w_ref[:, 2 * D:3 * D], dn_t,
                         preferred_element_type=jnp.float32)
    c = (lax.dot_general(es_ref[...], ps, dn,
                         preferred_element_type=jnp.float32)
         + lax.dot_general(er_ref[...], pr, dn,
                           preferred_element_type=jnp.float32)
         + lax.dot_general(eo_ref[...], po, dn,
                           preferred_element_type=jnp.float32))
    c_ref[...] = c + b_ref[...]


def _build_combined_table(subj_table, rel_table, obj_table, W, b):
    return pl.pallas_call(
        _fold_body,
        out_shape=jax.ShapeDtypeStruct((N_COMBO, D), jnp.float32),
    )(subj_table, rel_table, obj_table, W,
      jnp.asarray(_ES), jnp.asarray(_ER), jnp.asarray(_EO), b.reshape(1, D))


def _gather_kernel_fn(B, b_per_w, n_chunks, G):
    mesh = plsc.VectorSubcoreMesh(core_axis_name="c", subcore_axis_name="s")

    @functools.partial(
        pl.kernel,
        mesh=mesh,
        out_type=jax.ShapeDtypeStruct((B, D), jnp.float32),
        scratch_types=[
            pltpu.VMEM((b_per_w,), jnp.int32),
            pltpu.VMEM((b_per_w,), jnp.int32),
            pltpu.VMEM((b_per_w,), jnp.int32),
            pltpu.VMEM((n_chunks, G), jnp.int32),
            pltpu.VMEM((n_chunks, G, D), jnp.float32),
            pltpu.VMEM_SHARED((N_COMBO, D), jnp.float32),
            pltpu.SemaphoreType.DMA,
            pltpu.SemaphoreType.DMA,
            pltpu.SemaphoreType.DMA,
        ],
    )
    def k(s_hbm, r_hbm, o_hbm, table_hbm, out_hbm,
          s_v, r_v, o_v, j_v, rows_v, shared_tbl, gsem, osem, tsem):
        sid = lax.axis_index("s")
        wid = sid * SC_CORES + lax.axis_index("c")
        base = wid * b_per_w
        rows_per_sub = N_COMBO // SC_SUBCORES
        tcp = pltpu.async_copy(table_hbm.at[pl.ds(sid * rows_per_sub,
                                                  rows_per_sub)],
                               shared_tbl.at[pl.ds(sid * rows_per_sub,
                                                   rows_per_sub)], tsem)
        scp = pltpu.async_copy(s_hbm.at[pl.ds(base, b_per_w)], s_v, gsem)
        rcp = pltpu.async_copy(r_hbm.at[pl.ds(base, b_per_w)], r_v, gsem)
        ocp = pltpu.async_copy(o_hbm.at[pl.ds(base, b_per_w)], o_v, gsem)
        scp.wait()
        rcp.wait()
        ocp.wait()

        for c in range(n_chunks):
            @pl.loop(0, G, step=SC_LANES)
            def _(i, c=c):
                src = pl.ds(c * G + i, SC_LANES)
                j_v[c, pl.ds(i, SC_LANES)] = (
                    s_v[src] * (NS_REL * NS_OBJ)
                    + r_v[src] * NS_OBJ
                    + o_v[src])

        tcp.wait()
        plsc.subcore_barrier()

        gathers = [
            pltpu.async_copy(shared_tbl.at[j_v.at[c]], rows_v.at[c], gsem)
            for c in range(n_chunks)
        ]
        writes = []
        for c in range(n_chunks):
            gathers[c].wait()
            writes.append(pltpu.async_copy(
                rows_v.at[c], out_hbm.at[pl.ds(base + c * G, G)], osem))
        for w in writes:
            w.wait()

    return k


def kernel(s, r, o, subj_table, rel_table, obj_table, W, b):
    B = s.shape[0]
    b_per_w = B // NW
    G = 128
    n_chunks = b_per_w // G

    table = _build_combined_table(subj_table, rel_table, obj_table, W, b)
    gk = _gather_kernel_fn(B, b_per_w, n_chunks, G)
    return gk(s.astype(jnp.int32), r.astype(jnp.int32), o.astype(jnp.int32),
              table)


# speedup vs baseline: 6.8411x; 1.0334x over previous
import functools

import jax
import jax.numpy as jnp
import numpy as np
from jax import lax
from jax.experimental import pallas as pl
from jax.experimental.pallas import tpu as pltpu
from jax.experimental.pallas import tpu_sc as plsc

D = 128
NS_SUBJ = 16
NS_REL = 9
NS_OBJ = 16
N_COMBO = NS_SUBJ * NS_REL * NS_OBJ
STACK = NS_SUBJ + NS_REL + NS_OBJ
STACK_PAD = 48

SC_CORES = 2
SC_SUBCORES = 16
SC_LANES = 16
NW = SC_CORES * SC_SUBCORES


def _build_selectors():
    v = np.arange(N_COMBO)
    es = np.zeros((N_COMBO, NS_SUBJ), np.float32)
    er = np.zeros((N_COMBO, NS_REL), np.float32)
    eo = np.zeros((N_COMBO, NS_OBJ), np.float32)
    es[v, v // (NS_REL * NS_OBJ)] = 1.0
    er[v, (v // NS_OBJ) % NS_REL] = 1.0
    eo[v, v % NS_OBJ] = 1.0
    return es, er, eo


_ES, _ER, _EO = _build_selectors()


def _fold_body(subj_ref, rel_ref, obj_ref, w_ref, b_ref, c_ref):
    dn_t = (((1,), (1,)), ((), ()))
    ps = lax.dot_general(subj_ref[...], w_ref[:, 0:D], dn_t,
                         preferred_element_type=jnp.float32)
    pr = lax.dot_general(rel_ref[...], w_ref[:, D:2 * D], dn_t,
                         preferred_element_type=jnp.float32)
    po = lax.dot_general(obj_ref[...], w_ref[:, 2 * D:3 * D], dn_t,
                         preferred_element_type=jnp.float32)
    po = po + b_ref[...]
    c = (jnp.reshape(ps, (NS_SUBJ, 1, 1, D))
         + jnp.reshape(pr, (1, NS_REL, 1, D))
         + jnp.reshape(po, (1, 1, NS_OBJ, D)))
    c_ref[...] = jnp.reshape(
        jnp.broadcast_to(c, (NS_SUBJ, NS_REL, NS_OBJ, D)), (N_COMBO, D))


def _build_combined_table(subj_table, rel_table, obj_table, W, b):
    return pl.pallas_call(
        _fold_body,
        out_shape=jax.ShapeDtypeStruct((N_COMBO, D), jnp.float32),
    )(subj_table, rel_table, obj_table, W, b.reshape(1, D))


def _gather_kernel_fn(B, b_per_w, n_chunks, G):
    mesh = plsc.VectorSubcoreMesh(core_axis_name="c", subcore_axis_name="s")

    @functools.partial(
        pl.kernel,
        mesh=mesh,
        out_type=jax.ShapeDtypeStruct((B, D), jnp.float32),
        scratch_types=[
            pltpu.VMEM((b_per_w,), jnp.int32),
            pltpu.VMEM((b_per_w,), jnp.int32),
            pltpu.VMEM((b_per_w,), jnp.int32),
            pltpu.VMEM((n_chunks, G), jnp.int32),
            pltpu.VMEM((n_chunks, G, D), jnp.float32),
            pltpu.VMEM_SHARED((N_COMBO, D), jnp.float32),
            pltpu.SemaphoreType.DMA,
            pltpu.SemaphoreType.DMA,
            pltpu.SemaphoreType.DMA,
        ],
    )
    def k(s_hbm, r_hbm, o_hbm, table_hbm, out_hbm,
          s_v, r_v, o_v, j_v, rows_v, shared_tbl, gsem, osem, tsem):
        sid = lax.axis_index("s")
        wid = sid * SC_CORES + lax.axis_index("c")
        base = wid * b_per_w
        rows_per_sub = N_COMBO // SC_SUBCORES
        tcp = pltpu.async_copy(table_hbm.at[pl.ds(sid * rows_per_sub,
                                                  rows_per_sub)],
                               shared_tbl.at[pl.ds(sid * rows_per_sub,
                                                   rows_per_sub)], tsem)
        scp = pltpu.async_copy(s_hbm.at[pl.ds(base, b_per_w)], s_v, gsem)
        rcp = pltpu.async_copy(r_hbm.at[pl.ds(base, b_per_w)], r_v, gsem)
        ocp = pltpu.async_copy(o_hbm.at[pl.ds(base, b_per_w)], o_v, gsem)
        scp.wait()
        rcp.wait()
        ocp.wait()

        for c in range(n_chunks):
            @pl.loop(0, G, step=SC_LANES)
            def _(i, c=c):
                src = pl.ds(c * G + i, SC_LANES)
                j_v[c, pl.ds(i, SC_LANES)] = (
                    s_v[src] * (NS_REL * NS_OBJ)
                    + r_v[src] * NS_OBJ
                    + o_v[src])

        tcp.wait()
        plsc.subcore_barrier()

        gathers = [
            pltpu.async_copy(shared_tbl.at[j_v.at[c]], rows_v.at[c], gsem)
            for c in range(n_chunks)
        ]
        writes = []
        for c in range(n_chunks):
            gathers[c].wait()
            writes.append(pltpu.async_copy(
                rows_v.at[c], out_hbm.at[pl.ds(base + c * G, G)], osem))
        for w in writes:
            w.wait()

    return k


def kernel(s, r, o, subj_table, rel_table, obj_table, W, b):
    B = s.shape[0]
    b_per_w = B // NW
    G = 128
    n_chunks = b_per_w // G

    table = _build_combined_table(subj_table, rel_table, obj_table, W, b)
    gk = _gather_kernel_fn(B, b_per_w, n_chunks, G)
    return gk(s.astype(jnp.int32), r.astype(jnp.int32), o.astype(jnp.int32),
              table)
